# baseline (speedup 1.0000x reference)
import functools

import jax
import jax.numpy as jnp
from jax.experimental import pallas as pl
from jax.experimental.pallas import tpu as pltpu


def _stage_body(mask_ref, rw_ref, x_ref, w_ref, b_ref, o_ref,
                *, n_exp, relu_expert, relu_final):
    e = pl.program_id(0)
    y = jax.lax.dot_general(
        x_ref[...], w_ref[0],
        (((1,), (1,)), ((), ())),
        preferred_element_type=jnp.float32,
    )
    y = y + b_ref[0]
    if relu_expert:
        y = jnp.maximum(y, 0.0)
    m = mask_ref[0]
    coeff = (m[:, 0:1].astype(jnp.float32) * rw_ref[:, 0:1]
             + m[:, 1:2].astype(jnp.float32) * rw_ref[:, 1:2])
    contrib = y * coeff

    @pl.when(e == 0)
    def _():
        o_ref[...] = contrib

    @pl.when(e != 0)
    def _():
        o_ref[...] = o_ref[...] + contrib

    if relu_final:
        @pl.when(e == n_exp - 1)
        def _():
            o_ref[...] = jnp.maximum(o_ref[...], 0.0)


def _stage(x, mask, rw, W, b, *, relu_expert, relu_final):
    T, K = x.shape
    E, N, K2 = W.shape
    assert K == K2
    body = functools.partial(_stage_body, n_exp=E,
                             relu_expert=relu_expert, relu_final=relu_final)
    return pl.pallas_call(
        body,
        grid=(E,),
        in_specs=[
            pl.BlockSpec((1, T, 2), lambda e: (e, 0, 0)),
            pl.BlockSpec((T, 2), lambda e: (0, 0)),
            pl.BlockSpec((T, K), lambda e: (0, 0)),
            pl.BlockSpec((1, N, K), lambda e: (e, 0, 0)),
            pl.BlockSpec((1, 1, N), lambda e: (e, 0, 0)),
        ],
        out_specs=pl.BlockSpec((T, N), lambda e: (0, 0)),
        out_shape=jax.ShapeDtypeStruct((T, N), jnp.float32),
        compiler_params=pltpu.CompilerParams(
            dimension_semantics=("arbitrary",),
        ),
    )(mask.transpose(0, 2, 1), rw, x, W, b.reshape(E, 1, N))


def kernel(x, expert_mask1, expert_mask2, expert_mask3,
           routing_weights1, routing_weights2, routing_weights3,
           W1, b1, W2, b2, W3, b3):
    bsz, seq_len, hidden = x.shape
    T = bsz * seq_len
    xf = x.reshape(T, hidden)
    cs1 = _stage(xf, expert_mask1, routing_weights1, W1, b1,
                 relu_expert=False, relu_final=False)
    cs2 = _stage(cs1, expert_mask2, routing_weights2, W2, b2,
                 relu_expert=False, relu_final=False)
    out = _stage(cs2, expert_mask3, routing_weights3, W3, b3,
                 relu_expert=True, relu_final=True)
    return out.reshape(bsz, seq_len, -1)


# speedup vs baseline: 1.0547x; 1.0547x over previous
import functools

import jax
import jax.numpy as jnp
from jax.experimental import pallas as pl
from jax.experimental.pallas import tpu as pltpu


def _stage_body(mask_ref, rw_ref, x_ref, w_ref, b_ref, o_ref,
                *, n_exp, relu_expert, relu_final):
    e = pl.program_id(1)
    y = jax.lax.dot_general(
        x_ref[...], w_ref[0],
        (((1,), (1,)), ((), ())),
        preferred_element_type=jnp.float32,
    )
    y = y + b_ref[0]
    if relu_expert:
        y = jnp.maximum(y, 0.0)
    m = mask_ref[0]
    coeff = (m[0].astype(jnp.float32) * rw_ref[:, 0]
             + m[1].astype(jnp.float32) * rw_ref[:, 1])
    contrib = y * coeff[:, None]

    @pl.when(e == 0)
    def _():
        o_ref[...] = contrib

    @pl.when(e != 0)
    def _():
        o_ref[...] = o_ref[...] + contrib

    if relu_final:
        @pl.when(e == n_exp - 1)
        def _():
            o_ref[...] = jnp.maximum(o_ref[...], 0.0)


def _stage(x, mask, rw, W, b, *, relu_expert, relu_final, tm):
    T, K = x.shape
    E, N, K2 = W.shape
    assert K == K2 and T % tm == 0
    nt = T // tm
    body = functools.partial(_stage_body, n_exp=E,
                             relu_expert=relu_expert, relu_final=relu_final)
    return pl.pallas_call(
        body,
        grid=(nt, E),
        in_specs=[
            pl.BlockSpec((1, 2, tm), lambda i, e: (e, 0, i)),
            pl.BlockSpec((tm, 2), lambda i, e: (i, 0)),
            pl.BlockSpec((tm, K), lambda i, e: (i, 0)),
            pl.BlockSpec((1, N, K), lambda i, e: (e, 0, 0)),
            pl.BlockSpec((1, 1, N), lambda i, e: (e, 0, 0)),
        ],
        out_specs=pl.BlockSpec((tm, N), lambda i, e: (i, 0)),
        out_shape=jax.ShapeDtypeStruct((T, N), jnp.float32),
        compiler_params=pltpu.CompilerParams(
            dimension_semantics=("parallel", "arbitrary"),
        ),
    )(mask, rw, x, W, b.reshape(E, 1, N))


def kernel(x, expert_mask1, expert_mask2, expert_mask3,
           routing_weights1, routing_weights2, routing_weights3,
           W1, b1, W2, b2, W3, b3):
    bsz, seq_len, hidden = x.shape
    T = bsz * seq_len
    xf = x.reshape(T, hidden)
    cs1 = _stage(xf, expert_mask1, routing_weights1, W1, b1,
                 relu_expert=False, relu_final=False, tm=2048)
    cs2 = _stage(cs1, expert_mask2, routing_weights2, W2, b2,
                 relu_expert=False, relu_final=False, tm=2048)
    out = _stage(cs2, expert_mask3, routing_weights3, W3, b3,
                 relu_expert=True, relu_final=True, tm=2048)
    return out.reshape(bsz, seq_len, -1)


# speedup vs baseline: 1.0922x; 1.0356x over previous
import jax
import jax.numpy as jnp
from jax.experimental import pallas as pl
from jax.experimental.pallas import tpu as pltpu

_E = 8


def _coeff(m, rw_ref):
    return ((m[0].astype(jnp.float32) * rw_ref[:, 0]
             + m[1].astype(jnp.float32) * rw_ref[:, 1])[:, None])


def _mm(a, w):
    return jax.lax.dot_general(
        a, w, (((1,), (1,)), ((), ())),
        preferred_element_type=jnp.float32,
    )


def _fused12_body(m1_ref, rw1_ref, x_ref, w1_ref, b1_ref,
                  m2_ref, rw2_ref, w2_ref, b2_ref,
                  cs2_ref, cs1_ref):
    g = pl.program_id(0)

    @pl.when(g < _E)
    def _stage1():
        y = _mm(x_ref[...], w1_ref[0]) + b1_ref[0]
        contrib = y * _coeff(m1_ref[0], rw1_ref)

        @pl.when(g == 0)
        def _():
            cs1_ref[...] = contrib

        @pl.when(g > 0)
        def _():
            cs1_ref[...] = cs1_ref[...] + contrib

    @pl.when(g >= _E)
    def _stage2():
        y = _mm(cs1_ref[...], w2_ref[0]) + b2_ref[0]
        contrib = y * _coeff(m2_ref[0], rw2_ref)

        @pl.when(g == _E)
        def _():
            cs2_ref[...] = contrib

        @pl.when(g > _E)
        def _():
            cs2_ref[...] = cs2_ref[...] + contrib


def _stage3_body(m3_ref, rw3_ref, x_ref, w3_ref, b3_ref, o_ref):
    p = pl.program_id(0)
    y0 = jnp.maximum(_mm(x_ref[...], w3_ref[0]) + b3_ref[0], 0.0)
    y1 = jnp.maximum(_mm(x_ref[...], w3_ref[1]) + b3_ref[1], 0.0)
    contrib = (y0 * _coeff(m3_ref[0], rw3_ref)
               + y1 * _coeff(m3_ref[1], rw3_ref))

    @pl.when(p == 0)
    def _():
        o_ref[...] = contrib

    @pl.when(p > 0)
    def _():
        o_ref[...] = o_ref[...] + contrib

    @pl.when(p == _E // 2 - 1)
    def _():
        o_ref[...] = jnp.maximum(o_ref[...], 0.0)


def kernel(x, expert_mask1, expert_mask2, expert_mask3,
           routing_weights1, routing_weights2, routing_weights3,
           W1, b1, W2, b2, W3, b3):
    bsz, seq_len, hidden = x.shape
    T = bsz * seq_len
    xf = x.reshape(T, hidden)
    E, R0, H = W1.shape
    R1 = W2.shape[1]
    OUT = W3.shape[1]

    def e1(g):
        return jnp.clip(g, 0, _E - 1)

    def e2(g):
        return jnp.clip(g - _E, 0, _E - 1)

    cs2 = pl.pallas_call(
        _fused12_body,
        grid=(2 * _E,),
        in_specs=[
            pl.BlockSpec((1, 2, T), lambda g: (e1(g), 0, 0)),
            pl.BlockSpec((T, 2), lambda g: (0, 0)),
            pl.BlockSpec((T, H), lambda g: (0, 0)),
            pl.BlockSpec((1, R0, H), lambda g: (e1(g), 0, 0)),
            pl.BlockSpec((1, 1, R0), lambda g: (e1(g), 0, 0)),
            pl.BlockSpec((1, 2, T), lambda g: (e2(g), 0, 0)),
            pl.BlockSpec((T, 2), lambda g: (0, 0)),
            pl.BlockSpec((1, R1, R0), lambda g: (e2(g), 0, 0)),
            pl.BlockSpec((1, 1, R1), lambda g: (e2(g), 0, 0)),
        ],
        out_specs=pl.BlockSpec((T, R1), lambda g: (0, 0)),
        out_shape=jax.ShapeDtypeStruct((T, R1), jnp.float32),
        scratch_shapes=[pltpu.VMEM((T, R0), jnp.float32)],
        compiler_params=pltpu.CompilerParams(
            dimension_semantics=("arbitrary",),
        ),
    )(expert_mask1, routing_weights1, xf, W1, b1.reshape(E, 1, R0),
      expert_mask2, routing_weights2, W2, b2.reshape(E, 1, R1))

    out = pl.pallas_call(
        _stage3_body,
        grid=(_E // 2,),
        in_specs=[
            pl.BlockSpec((2, 2, T), lambda p: (p, 0, 0)),
            pl.BlockSpec((T, 2), lambda p: (0, 0)),
            pl.BlockSpec((T, R1), lambda p: (0, 0)),
            pl.BlockSpec((2, OUT, R1), lambda p: (p, 0, 0)),
            pl.BlockSpec((2, 1, OUT), lambda p: (p, 0, 0)),
        ],
        out_specs=pl.BlockSpec((T, OUT), lambda p: (0, 0)),
        out_shape=jax.ShapeDtypeStruct((T, OUT), jnp.float32),
        compiler_params=pltpu.CompilerParams(
            dimension_semantics=("arbitrary",),
        ),
    )(expert_mask3, routing_weights3, cs2, W3, b3.reshape(E, 1, OUT))

    return out.reshape(bsz, seq_len, OUT)


# speedup vs baseline: 1.1175x; 1.0231x over previous
import jax
import jax.numpy as jnp
from jax.experimental import pallas as pl
from jax.experimental.pallas import tpu as pltpu

_E = 8


def _coeff(m, rw_ref):
    return ((m[0].astype(jnp.float32) * rw_ref[:, 0]
             + m[1].astype(jnp.float32) * rw_ref[:, 1])[:, None])


def _mm(a, w):
    return jax.lax.dot_general(
        a, w, (((1,), (1,)), ((), ())),
        preferred_element_type=jnp.float32,
    )


def _fused12_body(m1_ref, rw1_ref, x_ref, w1_ref, b1_ref,
                  m2_ref, rw2_ref, w2_ref, b2_ref,
                  cs2_ref, cs1_ref):
    g = pl.program_id(0)

    @pl.when(g < _E)
    def _stage1():
        y = _mm(x_ref[...], w1_ref[0]) + b1_ref[0]
        contrib = y * _coeff(m1_ref[0], rw1_ref)

        @pl.when(g == 0)
        def _():
            cs1_ref[...] = contrib

        @pl.when(g > 0)
        def _():
            cs1_ref[...] = cs1_ref[...] + contrib

    @pl.when(g >= _E)
    def _stage2():
        y0 = _mm(cs1_ref[...], w2_ref[0]) + b2_ref[0]
        y1 = _mm(cs1_ref[...], w2_ref[1]) + b2_ref[1]
        contrib = (y0 * _coeff(m2_ref[0], rw2_ref)
                   + y1 * _coeff(m2_ref[1], rw2_ref))

        @pl.when(g == _E)
        def _():
            cs2_ref[...] = contrib

        @pl.when(g > _E)
        def _():
            cs2_ref[...] = cs2_ref[...] + contrib


def _stage3_body(m3_ref, rw3_ref, x_ref, w3_ref, b3_ref, o_ref):
    p = pl.program_id(0)
    y0 = jnp.maximum(_mm(x_ref[...], w3_ref[0]) + b3_ref[0], 0.0)
    y1 = jnp.maximum(_mm(x_ref[...], w3_ref[1]) + b3_ref[1], 0.0)
    contrib = (y0 * _coeff(m3_ref[0], rw3_ref)
               + y1 * _coeff(m3_ref[1], rw3_ref))

    @pl.when(p == 0)
    def _():
        o_ref[...] = contrib

    @pl.when(p > 0)
    def _():
        o_ref[...] = o_ref[...] + contrib

    @pl.when(p == _E // 2 - 1)
    def _():
        o_ref[...] = jnp.maximum(o_ref[...], 0.0)


def kernel(x, expert_mask1, expert_mask2, expert_mask3,
           routing_weights1, routing_weights2, routing_weights3,
           W1, b1, W2, b2, W3, b3):
    bsz, seq_len, hidden = x.shape
    T = bsz * seq_len
    xf = x.reshape(T, hidden)
    E, R0, H = W1.shape
    R1 = W2.shape[1]
    OUT = W3.shape[1]

    def e1(g):
        return jnp.clip(g, 0, _E - 1)

    def e2(g):
        return jnp.clip(g - _E, 0, _E // 2 - 1)

    cs2 = pl.pallas_call(
        _fused12_body,
        grid=(_E + _E // 2,),
        in_specs=[
            pl.BlockSpec((1, 2, T), lambda g: (e1(g), 0, 0)),
            pl.BlockSpec((T, 2), lambda g: (0, 0)),
            pl.BlockSpec((T, H), lambda g: (0, 0)),
            pl.BlockSpec((1, R0, H), lambda g: (e1(g), 0, 0)),
            pl.BlockSpec((1, 1, R0), lambda g: (e1(g), 0, 0)),
            pl.BlockSpec((2, 2, T), lambda g: (e2(g), 0, 0)),
            pl.BlockSpec((T, 2), lambda g: (0, 0)),
            pl.BlockSpec((2, R1, R0), lambda g: (e2(g), 0, 0)),
            pl.BlockSpec((2, 1, R1), lambda g: (e2(g), 0, 0)),
        ],
        out_specs=pl.BlockSpec((T, R1), lambda g: (0, 0)),
        out_shape=jax.ShapeDtypeStruct((T, R1), jnp.float32),
        scratch_shapes=[pltpu.VMEM((T, R0), jnp.float32)],
        compiler_params=pltpu.CompilerParams(
            dimension_semantics=("arbitrary",),
        ),
    )(expert_mask1, routing_weights1, xf, W1, b1.reshape(E, 1, R0),
      expert_mask2, routing_weights2, W2, b2.reshape(E, 1, R1))

    out = pl.pallas_call(
        _stage3_body,
        grid=(_E // 2,),
        in_specs=[
            pl.BlockSpec((2, 2, T), lambda p: (p, 0, 0)),
            pl.BlockSpec((T, 2), lambda p: (0, 0)),
            pl.BlockSpec((T, R1), lambda p: (0, 0)),
            pl.BlockSpec((2, OUT, R1), lambda p: (p, 0, 0)),
            pl.BlockSpec((2, 1, OUT), lambda p: (p, 0, 0)),
        ],
        out_specs=pl.BlockSpec((T, OUT), lambda p: (0, 0)),
        out_shape=jax.ShapeDtypeStruct((T, OUT), jnp.float32),
        compiler_params=pltpu.CompilerParams(
            dimension_semantics=("arbitrary",),
        ),
    )(expert_mask3, routing_weights3, cs2, W3, b3.reshape(E, 1, OUT))

    return out.reshape(bsz, seq_len, OUT)
